# Initial kernel scaffold; baseline (speedup 1.0000x reference)
import jax
import jax.numpy as jnp
from jax.experimental import pallas as pl


def kernel(seqs, species, vocab_emb, cat_emb):
    raise NotImplementedError("write your pallas kernel here")


# speedup vs baseline: 6.3272x; 6.3272x over previous
import functools

import jax
import jax.numpy as jnp
from jax import lax
from jax.experimental import pallas as pl
from jax.experimental.pallas import tpu as pltpu
from jax.experimental.pallas import tpu_sc as plsc

EMB = 64
VOCAB = 5
PAD_IDX = VOCAB - 1
BATCH = 4096
SEQ = 200

BB = 128


EMB_PAD = 128


def _sc_species_gather(cat_emb_pad, species):
    info = plsc.get_sparse_core_info()
    nc, ns = info.num_cores, info.num_subcores
    nw = nc * ns
    b_per_w = BATCH // nw

    mesh = plsc.VectorSubcoreMesh(core_axis_name="c", subcore_axis_name="s")

    @functools.partial(
        pl.kernel,
        mesh=mesh,
        out_type=jax.ShapeDtypeStruct((BATCH, EMB_PAD), jnp.float32),
        scratch_types=[
            pltpu.VMEM((b_per_w,), jnp.int32),
            pltpu.VMEM((b_per_w, EMB_PAD), jnp.float32),
            pltpu.SemaphoreType.DMA,
        ],
    )
    def gather_kernel(table_hbm, idx_hbm, out_hbm, idx_v, rows_v, sem):
        wid = lax.axis_index("s") * nc + lax.axis_index("c")
        base = wid * b_per_w
        pltpu.sync_copy(idx_hbm.at[pl.ds(base, b_per_w)], idx_v)
        pltpu.async_copy(table_hbm.at[idx_v], rows_v, sem).wait()
        pltpu.sync_copy(rows_v, out_hbm.at[pl.ds(base, b_per_w)])

    return gather_kernel(cat_emb_pad, species)


def _tc_body(seqs_ref, spemb_ref, vocab_ref, total_ref, spout_ref):
    s = seqs_ref[...]
    spe = spemb_ref[:, :EMB]
    vt = vocab_ref[...]
    spout_ref[...] = spe
    acc = jnp.broadcast_to(spe[:, None, :], (BB, SEQ, EMB))
    s3 = s[:, :, None]
    for v in range(VOCAB - 1):
        acc = acc + jnp.where(s3 == v, vt[v], 0.0)
    total_ref[...] = acc


def _tc_fused(seqs, spemb_pad, vocab_emb):
    nb = BATCH // BB
    return pl.pallas_call(
        _tc_body,
        grid=(nb,),
        in_specs=[
            pl.BlockSpec((BB, SEQ), lambda i: (i, 0)),
            pl.BlockSpec((BB, EMB_PAD), lambda i: (i, 0)),
            pl.BlockSpec((VOCAB, EMB), lambda i: (0, 0)),
        ],
        out_specs=[
            pl.BlockSpec((BB, SEQ, EMB), lambda i: (i, 0, 0)),
            pl.BlockSpec((BB, EMB), lambda i: (i, 0)),
        ],
        out_shape=[
            jax.ShapeDtypeStruct((BATCH, SEQ, EMB), jnp.float32),
            jax.ShapeDtypeStruct((BATCH, EMB), jnp.float32),
        ],
    )(seqs, spemb_pad, vocab_emb)


def kernel(seqs, species, vocab_emb, cat_emb):
    seqs = seqs.astype(jnp.int32)
    species = species.astype(jnp.int32)
    cat_emb_pad = jnp.pad(cat_emb, ((0, 0), (0, EMB_PAD - EMB)))
    spemb_pad = _sc_species_gather(cat_emb_pad, species)
    total, spemb = _tc_fused(seqs, spemb_pad, vocab_emb)
    return total, spemb


# speedup vs baseline: 6.8357x; 1.0804x over previous
import functools

import jax
import jax.numpy as jnp
from jax import lax
from jax.experimental import pallas as pl
from jax.experimental.pallas import tpu as pltpu
from jax.experimental.pallas import tpu_sc as plsc

EMB = 64
VOCAB = 5
PAD_IDX = VOCAB - 1
BATCH = 4096
SEQ = 200

BB = 128


EMB_PAD = 128


def _sc_species_gather(cat_emb_pad, species):
    info = plsc.get_sparse_core_info()
    nc, ns = info.num_cores, info.num_subcores
    nw = nc * ns
    b_per_w = BATCH // nw

    mesh = plsc.VectorSubcoreMesh(core_axis_name="c", subcore_axis_name="s")

    @functools.partial(
        pl.kernel,
        mesh=mesh,
        out_type=jax.ShapeDtypeStruct((BATCH, EMB_PAD), jnp.float32),
        scratch_types=[
            pltpu.VMEM((b_per_w,), jnp.int32),
            pltpu.VMEM((b_per_w, EMB_PAD), jnp.float32),
            pltpu.SemaphoreType.DMA,
        ],
    )
    def gather_kernel(table_hbm, idx_hbm, out_hbm, idx_v, rows_v, sem):
        wid = lax.axis_index("s") * nc + lax.axis_index("c")
        base = wid * b_per_w
        pltpu.sync_copy(idx_hbm.at[pl.ds(base, b_per_w)], idx_v)
        pltpu.async_copy(table_hbm.at[idx_v], rows_v, sem).wait()
        pltpu.sync_copy(rows_v, out_hbm.at[pl.ds(base, b_per_w)])

    return gather_kernel(cat_emb_pad, species)


def _tc_body(seqs_ref, spemb_ref, vocab_ref, total_ref, spout_ref):
    s = seqs_ref[...]
    spe = spemb_ref[:, :EMB]
    vt = vocab_ref[...]
    row = lax.broadcasted_iota(jnp.int32, (VOCAB, 1), 0)
    vt = jnp.where(row == PAD_IDX, 0.0, vt)
    spout_ref[...] = spe
    idx = jnp.broadcast_to(s[:, :, None], (BB, SEQ, EMB))
    xt = jnp.broadcast_to(vt[None], (BB, VOCAB, EMB))
    seq_emb = jnp.take_along_axis(xt, idx, axis=1, mode="promise_in_bounds")
    total_ref[...] = seq_emb + spe[:, None, :]


def _tc_fused(seqs, spemb_pad, vocab_emb):
    nb = BATCH // BB
    return pl.pallas_call(
        _tc_body,
        grid=(nb,),
        in_specs=[
            pl.BlockSpec((BB, SEQ), lambda i: (i, 0)),
            pl.BlockSpec((BB, EMB_PAD), lambda i: (i, 0)),
            pl.BlockSpec((VOCAB, EMB), lambda i: (0, 0)),
        ],
        out_specs=[
            pl.BlockSpec((BB, SEQ, EMB), lambda i: (i, 0, 0)),
            pl.BlockSpec((BB, EMB), lambda i: (i, 0)),
        ],
        out_shape=[
            jax.ShapeDtypeStruct((BATCH, SEQ, EMB), jnp.float32),
            jax.ShapeDtypeStruct((BATCH, EMB), jnp.float32),
        ],
    )(seqs, spemb_pad, vocab_emb)


def kernel(seqs, species, vocab_emb, cat_emb):
    seqs = seqs.astype(jnp.int32)
    species = species.astype(jnp.int32)
    cat_emb_pad = jnp.pad(cat_emb, ((0, 0), (0, EMB_PAD - EMB)))
    spemb_pad = _sc_species_gather(cat_emb_pad, species)
    total, spemb = _tc_fused(seqs, spemb_pad, vocab_emb)
    return total, spemb
